# Initial kernel scaffold; baseline (speedup 1.0000x reference)
import jax
import jax.numpy as jnp
from jax.experimental import pallas as pl


def kernel(scores, boxes, classes):
    raise NotImplementedError("write your pallas kernel here")


# speedup vs baseline: 8.0969x; 8.0969x over previous
import functools

import jax
import jax.numpy as jnp
from jax import lax
from jax.experimental import pallas as pl
from jax.experimental.pallas import tpu as pltpu
from jax.experimental.pallas import tpu_sc as plsc

LANES = 16
NSUB = 16
MAX_OUT = 100
IOU_THR = 0.5
SCORE_THR = 0.1
NEG = -1.0e30
BIGI = 2**30
BIGF = 1.0e30


def _nms_body(per, nvec, s_hbm, y1_hbm, x1_hbm, y2_hbm, x2_hbm, c_hbm,
              out_hbm, work, wy1, wx1, wy2, wx2, warea, wcls, stage, buf,
              rec, shared):
    sid = lax.axis_index("s")
    base = sid * per
    lane = lax.iota(jnp.int32, LANES)

    pltpu.sync_copy(s_hbm.at[pl.ds(base, per)], work)
    pltpu.sync_copy(y1_hbm.at[pl.ds(base, per)], wy1)
    pltpu.sync_copy(x1_hbm.at[pl.ds(base, per)], wx1)
    pltpu.sync_copy(y2_hbm.at[pl.ds(base, per)], wy2)
    pltpu.sync_copy(x2_hbm.at[pl.ds(base, per)], wx2)
    pltpu.sync_copy(c_hbm.at[pl.ds(base, per)], wcls)

    def init_body(j, carry):
        maxv, maxi = carry
        off = j * LANES
        v = work[pl.ds(off, LANES)]
        w = jnp.where(v >= SCORE_THR, v, NEG)
        work[pl.ds(off, LANES)] = w
        a = (wy2[pl.ds(off, LANES)] - wy1[pl.ds(off, LANES)]) * (
            wx2[pl.ds(off, LANES)] - wx1[pl.ds(off, LANES)])
        warea[pl.ds(off, LANES)] = a
        idxv = off + lane
        upd = w > maxv
        return jnp.where(upd, w, maxv), jnp.where(upd, idxv, maxi)

    maxv0 = jnp.full((LANES,), NEG, jnp.float32)
    maxi0 = jnp.zeros((LANES,), jnp.int32)
    maxv, maxi = lax.fori_loop(0, nvec, init_body, (maxv0, maxi0))

    def round_body(i, carry):
        maxv, maxi = carry
        m = jnp.max(maxv)
        jloc = jnp.min(jnp.where(maxv == m, maxi, BIGI))
        jidx = jnp.full((LANES,), jloc, jnp.int32)
        y1g = plsc.load_gather(wy1, [jidx])
        x1g = plsc.load_gather(wx1, [jidx])
        y2g = plsc.load_gather(wy2, [jidx])
        x2g = plsc.load_gather(wx2, [jidx])
        cg = plsc.load_gather(wcls, [jidx])
        gidxf = (jloc + base).astype(jnp.float32)
        r = jnp.where(lane == 0, m, 0.0)
        r = jnp.where(lane == 1, gidxf, r)
        r = jnp.where(lane == 2, y1g, r)
        r = jnp.where(lane == 3, x1g, r)
        r = jnp.where(lane == 4, y2g, r)
        r = jnp.where(lane == 5, x2g, r)
        r = jnp.where(lane == 6, cg, r)
        stage[...] = r
        pltpu.sync_copy(stage, shared.at[pl.ds(sid * LANES, LANES)])
        plsc.subcore_barrier()
        pltpu.sync_copy(shared, buf)
        plsc.subcore_barrier()

        row = lane * LANES
        maxes = plsc.load_gather(buf, [row])
        idxs = plsc.load_gather(buf, [row + 1])
        m2 = jnp.max(maxes)
        ok = m2 >= SCORE_THR
        cand = jnp.where(maxes == m2, idxs, BIGF)
        widxf = jnp.min(cand)
        lm = cand == widxf
        y1s = jnp.min(jnp.where(lm, plsc.load_gather(buf, [row + 2]), BIGF))
        x1s = jnp.min(jnp.where(lm, plsc.load_gather(buf, [row + 3]), BIGF))
        y2s = jnp.min(jnp.where(lm, plsc.load_gather(buf, [row + 4]), BIGF))
        x2s = jnp.min(jnp.where(lm, plsc.load_gather(buf, [row + 5]), BIGF))
        cs = jnp.min(jnp.where(lm, plsc.load_gather(buf, [row + 6]), BIGF))
        area_s = (y2s - y1s) * (x2s - x1s)
        widxi = widxf.astype(jnp.int32) - base

        rv = jnp.where(lane == 0, m2, 0.0)
        rv = jnp.where(lane == 1, y1s, rv)
        rv = jnp.where(lane == 2, x1s, rv)
        rv = jnp.where(lane == 3, y2s, rv)
        rv = jnp.where(lane == 4, x2s, rv)
        rv = jnp.where(lane == 5, cs, rv)
        rec[pl.ds(i * LANES, LANES)] = jnp.where(ok, rv, 0.0)

        def fuse(j, c2):
            mv, mi = c2
            off = j * LANES
            sl = pl.ds(off, LANES)
            w = work[sl]
            iy1 = jnp.maximum(wy1[sl], y1s)
            ix1 = jnp.maximum(wx1[sl], x1s)
            iy2 = jnp.minimum(wy2[sl], y2s)
            ix2 = jnp.minimum(wx2[sl], x2s)
            inter = jnp.maximum(iy2 - iy1, 0.0) * jnp.maximum(ix2 - ix1, 0.0)
            union = (warea[sl] + area_s) - inter
            idxv = off + lane
            kill = ((inter + inter > union) | (idxv == widxi)) & ok
            w2 = jnp.where(kill, NEG, w)
            work[sl] = w2
            upd = w2 > mv
            return jnp.where(upd, w2, mv), jnp.where(upd, idxv, mi)

        return lax.fori_loop(0, nvec, fuse, (maxv0, maxi0))

    lax.fori_loop(0, MAX_OUT, round_body, (maxv, maxi))

    @pl.when(sid == 0)
    def _():
        pltpu.sync_copy(rec, out_hbm)


@jax.jit
def kernel(scores, boxes, classes):
    n = scores.shape[0]
    nvec = -(-n // (NSUB * LANES))
    per = nvec * LANES
    npad = per * NSUB
    pad = npad - n

    f32 = jnp.float32
    s_p = jnp.concatenate([scores.astype(f32), jnp.full((pad,), -1.0, f32)])
    y1_p = jnp.concatenate([boxes[:, 0].astype(f32), jnp.zeros((pad,), f32)])
    x1_p = jnp.concatenate([boxes[:, 1].astype(f32), jnp.zeros((pad,), f32)])
    y2_p = jnp.concatenate([boxes[:, 2].astype(f32), jnp.zeros((pad,), f32)])
    x2_p = jnp.concatenate([boxes[:, 3].astype(f32), jnp.zeros((pad,), f32)])
    c_p = jnp.concatenate([classes.astype(f32), jnp.zeros((pad,), f32)])

    mesh = plsc.VectorSubcoreMesh(
        core_axis_name="c", subcore_axis_name="s", num_cores=1,
        num_subcores=NSUB)

    run = pl.kernel(
        functools.partial(_nms_body, per, nvec),
        out_type=jax.ShapeDtypeStruct((MAX_OUT * LANES,), f32),
        mesh=mesh,
        compiler_params=pltpu.CompilerParams(needs_layout_passes=False),
        scratch_types=[
            pltpu.VMEM((per,), f32),
            pltpu.VMEM((per,), f32),
            pltpu.VMEM((per,), f32),
            pltpu.VMEM((per,), f32),
            pltpu.VMEM((per,), f32),
            pltpu.VMEM((per,), f32),
            pltpu.VMEM((per,), f32),
            pltpu.VMEM((LANES,), f32),
            pltpu.VMEM((NSUB * LANES,), f32),
            pltpu.VMEM((MAX_OUT * LANES,), f32),
            pltpu.VMEM_SHARED((NSUB * LANES,), f32),
        ],
    )
    rec = run(s_p, y1_p, x1_p, y2_p, x2_p, c_p).reshape(MAX_OUT, LANES)
    return rec[:, 0], rec[:, 1:5], rec[:, 5]


# speedup vs baseline: 12.2372x; 1.5113x over previous
import functools

import jax
import jax.numpy as jnp
from jax import lax
from jax.experimental import pallas as pl
from jax.experimental.pallas import tpu as pltpu
from jax.experimental.pallas import tpu_sc as plsc

LANES = 16
NSUB = 16
MAX_OUT = 100
IOU_THR = 0.5
SCORE_THR = 0.1
NEG = -1.0e30
BIGI = 2**30
BIGF = 1.0e30


def _nms_body(per, nvec, s_hbm, y1_hbm, x1_hbm, y2_hbm, x2_hbm, c_hbm,
              out_hbm, work, wy1, wx1, wy2, wx2, warea, wcls, stage, buf,
              rec, shared):
    sid = lax.axis_index("s")
    base = sid * per
    lane = lax.iota(jnp.int32, LANES)

    pltpu.sync_copy(s_hbm.at[pl.ds(base, per)], work)
    pltpu.sync_copy(y1_hbm.at[pl.ds(base, per)], wy1)
    pltpu.sync_copy(x1_hbm.at[pl.ds(base, per)], wx1)
    pltpu.sync_copy(y2_hbm.at[pl.ds(base, per)], wy2)
    pltpu.sync_copy(x2_hbm.at[pl.ds(base, per)], wx2)
    pltpu.sync_copy(c_hbm.at[pl.ds(base, per)], wcls)

    def init_body(j, carry):
        maxv, maxi = carry
        off = j * LANES
        v = work[pl.ds(off, LANES)]
        w = jnp.where(v >= SCORE_THR, v, NEG)
        work[pl.ds(off, LANES)] = w
        a = (wy2[pl.ds(off, LANES)] - wy1[pl.ds(off, LANES)]) * (
            wx2[pl.ds(off, LANES)] - wx1[pl.ds(off, LANES)])
        warea[pl.ds(off, LANES)] = a
        idxv = off + lane
        upd = (w > maxv) | ((w == maxv) & (idxv < maxi))
        return jnp.where(upd, w, maxv), jnp.where(upd, idxv, maxi)

    maxv0 = jnp.full((LANES,), NEG, jnp.float32)
    maxi0 = jnp.zeros((LANES,), jnp.int32)
    maxv, maxi = lax.fori_loop(0, nvec, init_body, (maxv0, maxi0))

    def round_body(i, carry):
        maxv, maxi = carry
        m = jnp.max(maxv)
        jloc = jnp.min(jnp.where(maxv == m, maxi, BIGI))
        jidx = jnp.full((LANES,), jloc, jnp.int32)
        y1g = plsc.load_gather(wy1, [jidx])
        x1g = plsc.load_gather(wx1, [jidx])
        y2g = plsc.load_gather(wy2, [jidx])
        x2g = plsc.load_gather(wx2, [jidx])
        cg = plsc.load_gather(wcls, [jidx])
        gidxf = (jloc + base).astype(jnp.float32)
        r = jnp.where(lane == 0, m, 0.0)
        r = jnp.where(lane == 1, gidxf, r)
        r = jnp.where(lane == 2, y1g, r)
        r = jnp.where(lane == 3, x1g, r)
        r = jnp.where(lane == 4, y2g, r)
        r = jnp.where(lane == 5, x2g, r)
        r = jnp.where(lane == 6, cg, r)
        stage[...] = r
        pltpu.sync_copy(stage, shared.at[pl.ds(sid * LANES, LANES)])
        plsc.subcore_barrier()
        pltpu.sync_copy(shared, buf)
        plsc.subcore_barrier()

        row = lane * LANES
        maxes = plsc.load_gather(buf, [row])
        idxs = plsc.load_gather(buf, [row + 1])
        m2 = jnp.max(maxes)
        ok = m2 >= SCORE_THR
        cand = jnp.where(maxes == m2, idxs, BIGF)
        widxf = jnp.min(cand)
        lm = cand == widxf
        y1s = jnp.min(jnp.where(lm, plsc.load_gather(buf, [row + 2]), BIGF))
        x1s = jnp.min(jnp.where(lm, plsc.load_gather(buf, [row + 3]), BIGF))
        y2s = jnp.min(jnp.where(lm, plsc.load_gather(buf, [row + 4]), BIGF))
        x2s = jnp.min(jnp.where(lm, plsc.load_gather(buf, [row + 5]), BIGF))
        cs = jnp.min(jnp.where(lm, plsc.load_gather(buf, [row + 6]), BIGF))
        area_s = (y2s - y1s) * (x2s - x1s)
        widxi = widxf.astype(jnp.int32) - base

        rv = jnp.where(lane == 0, m2, 0.0)
        rv = jnp.where(lane == 1, y1s, rv)
        rv = jnp.where(lane == 2, x1s, rv)
        rv = jnp.where(lane == 3, y2s, rv)
        rv = jnp.where(lane == 4, x2s, rv)
        rv = jnp.where(lane == 5, cs, rv)
        rec[pl.ds(i * LANES, LANES)] = jnp.where(ok, rv, 0.0)

        @plsc.parallel_loop(0, nvec, unroll=4, carry=(maxv0, maxi0))
        def fuse(j, c2):
            mv, mi = c2
            off = j * LANES
            sl = pl.ds(off, LANES)
            w = work[sl]
            iy1 = jnp.maximum(wy1[sl], y1s)
            ix1 = jnp.maximum(wx1[sl], x1s)
            iy2 = jnp.minimum(wy2[sl], y2s)
            ix2 = jnp.minimum(wx2[sl], x2s)
            inter = jnp.maximum(iy2 - iy1, 0.0) * jnp.maximum(ix2 - ix1, 0.0)
            union = (warea[sl] + area_s) - inter
            idxv = off + lane
            kill = ((inter + inter > union) | (idxv == widxi)) & ok
            w2 = jnp.where(kill, NEG, w)
            work[sl] = w2
            upd = (w2 > mv) | ((w2 == mv) & (idxv < mi))
            return jnp.where(upd, w2, mv), jnp.where(upd, idxv, mi)

        return fuse

    lax.fori_loop(0, MAX_OUT, round_body, (maxv, maxi))

    @pl.when(sid == 0)
    def _():
        pltpu.sync_copy(rec, out_hbm)


@jax.jit
def kernel(scores, boxes, classes):
    n = scores.shape[0]
    nvec = -(-n // (NSUB * LANES))
    per = nvec * LANES
    npad = per * NSUB
    pad = npad - n

    f32 = jnp.float32
    s_p = jnp.concatenate([scores.astype(f32), jnp.full((pad,), -1.0, f32)])
    y1_p = jnp.concatenate([boxes[:, 0].astype(f32), jnp.zeros((pad,), f32)])
    x1_p = jnp.concatenate([boxes[:, 1].astype(f32), jnp.zeros((pad,), f32)])
    y2_p = jnp.concatenate([boxes[:, 2].astype(f32), jnp.zeros((pad,), f32)])
    x2_p = jnp.concatenate([boxes[:, 3].astype(f32), jnp.zeros((pad,), f32)])
    c_p = jnp.concatenate([classes.astype(f32), jnp.zeros((pad,), f32)])

    mesh = plsc.VectorSubcoreMesh(
        core_axis_name="c", subcore_axis_name="s", num_cores=1,
        num_subcores=NSUB)

    run = pl.kernel(
        functools.partial(_nms_body, per, nvec),
        out_type=jax.ShapeDtypeStruct((MAX_OUT * LANES,), f32),
        mesh=mesh,
        compiler_params=pltpu.CompilerParams(needs_layout_passes=False),
        scratch_types=[
            pltpu.VMEM((per,), f32),
            pltpu.VMEM((per,), f32),
            pltpu.VMEM((per,), f32),
            pltpu.VMEM((per,), f32),
            pltpu.VMEM((per,), f32),
            pltpu.VMEM((per,), f32),
            pltpu.VMEM((per,), f32),
            pltpu.VMEM((LANES,), f32),
            pltpu.VMEM((NSUB * LANES,), f32),
            pltpu.VMEM((MAX_OUT * LANES,), f32),
            pltpu.VMEM_SHARED((NSUB * LANES,), f32),
        ],
    )
    rec = run(s_p, y1_p, x1_p, y2_p, x2_p, c_p).reshape(MAX_OUT, LANES)
    return rec[:, 0], rec[:, 1:5], rec[:, 5]


# speedup vs baseline: 12.4053x; 1.0137x over previous
import functools

import jax
import jax.numpy as jnp
from jax import lax
from jax.experimental import pallas as pl
from jax.experimental.pallas import tpu as pltpu
from jax.experimental.pallas import tpu_sc as plsc

LANES = 16
NSUB = 16
MAX_OUT = 100
IOU_THR = 0.5
SCORE_THR = 0.1
NEG = -1.0e30
BIGI = 2**30
BIGF = 1.0e30
ROW = 16


def _nms_body(per, nvec, s_hbm, y1_hbm, x1_hbm, y2_hbm, x2_hbm, c_hbm,
              out_hbm, work, wy1, wx1, wy2, wx2, warea, wcls, stage, buf,
              rec, shared):
    sid = lax.axis_index("s")
    base = sid * per
    lane = lax.iota(jnp.int32, LANES)

    pltpu.sync_copy(s_hbm.at[pl.ds(base, per)], work)
    pltpu.sync_copy(y1_hbm.at[pl.ds(base, per)], wy1)
    pltpu.sync_copy(x1_hbm.at[pl.ds(base, per)], wx1)
    pltpu.sync_copy(y2_hbm.at[pl.ds(base, per)], wy2)
    pltpu.sync_copy(x2_hbm.at[pl.ds(base, per)], wx2)
    pltpu.sync_copy(c_hbm.at[pl.ds(base, per)], wcls)

    def init_body(j, carry):
        maxv, maxi = carry
        off = j * LANES
        v = work[pl.ds(off, LANES)]
        w = jnp.where(v >= SCORE_THR, v, NEG)
        work[pl.ds(off, LANES)] = w
        a = (wy2[pl.ds(off, LANES)] - wy1[pl.ds(off, LANES)]) * (
            wx2[pl.ds(off, LANES)] - wx1[pl.ds(off, LANES)])
        warea[pl.ds(off, LANES)] = a
        idxv = off + lane
        upd = (w > maxv) | ((w == maxv) & (idxv < maxi))
        return jnp.where(upd, w, maxv), jnp.where(upd, idxv, maxi)

    maxv0 = jnp.full((LANES,), NEG, jnp.float32)
    maxi0 = jnp.zeros((LANES,), jnp.int32)
    maxv, maxi = lax.fori_loop(0, nvec, init_body, (maxv0, maxi0))

    def round_body(i, carry):
        maxv, maxi = carry
        par = (i % 2) * (NSUB * ROW)
        m = jnp.max(maxv)
        jloc = jnp.min(jnp.where(maxv == m, maxi, BIGI))
        jidx = jnp.full((LANES,), jloc, jnp.int32)
        y1g = plsc.load_gather(wy1, [jidx])
        x1g = plsc.load_gather(wx1, [jidx])
        y2g = plsc.load_gather(wy2, [jidx])
        x2g = plsc.load_gather(wx2, [jidx])
        cg = plsc.load_gather(wcls, [jidx])
        gidxf = (jloc + base).astype(jnp.float32)
        r = jnp.where(lane == 0, m, 0.0)
        r = jnp.where(lane == 1, gidxf, r)
        r = jnp.where(lane == 2, y1g, r)
        r = jnp.where(lane == 3, x1g, r)
        r = jnp.where(lane == 4, y2g, r)
        r = jnp.where(lane == 5, x2g, r)
        r = jnp.where(lane == 6, cg, r)
        stage[...] = r
        pltpu.sync_copy(stage, shared.at[pl.ds(par + sid * ROW, LANES)])
        plsc.subcore_barrier()
        pltpu.sync_copy(shared.at[pl.ds(par, NSUB * ROW)], buf)

        row = lane * ROW
        maxes = plsc.load_gather(buf, [row])
        idxs = plsc.load_gather(buf, [row + 1])
        m2 = jnp.max(maxes)
        ok = m2 >= SCORE_THR
        cand = jnp.where(maxes == m2, idxs, BIGF)
        widxf = jnp.min(cand)
        widx = widxf.astype(jnp.int32)
        t = widx // per
        trow = jnp.full((LANES,), t * ROW, jnp.int32)
        y1s = plsc.load_gather(buf, [trow + 2])
        x1s = plsc.load_gather(buf, [trow + 3])
        y2s = plsc.load_gather(buf, [trow + 4])
        x2s = plsc.load_gather(buf, [trow + 5])
        cs = plsc.load_gather(buf, [trow + 6])

        rv = jnp.where(lane == 0, m2, 0.0)
        rv = jnp.where(lane == 1, y1s, rv)
        rv = jnp.where(lane == 2, x1s, rv)
        rv = jnp.where(lane == 3, y2s, rv)
        rv = jnp.where(lane == 4, x2s, rv)
        rv = jnp.where(lane == 5, cs, rv)
        rec[pl.ds(i * LANES, LANES)] = jnp.where(ok, rv, 0.0)

        widxi = widx - base
        kmask = ok & (t == sid) & (lane == 0)
        kaddr = jnp.clip(jnp.full((LANES,), widxi, jnp.int32), 0, per - 1)
        plsc.store_scatter(work, [kaddr], jnp.full((LANES,), NEG, jnp.float32),
                           mask=kmask)
        y1k = jnp.where(ok, y1s, BIGF)
        x1k = jnp.where(ok, x1s, BIGF)
        y2k = jnp.where(ok, y2s, BIGF)
        x2k = jnp.where(ok, x2s, BIGF)
        area_k = (y2k - y1k) * (x2k - x1k)

        @plsc.parallel_loop(0, nvec, unroll=8, carry=(maxv0, maxi0))
        def fuse(j, c2):
            mv, mi = c2
            off = j * LANES
            sl = pl.ds(off, LANES)
            w = work[sl]
            iy1 = jnp.maximum(wy1[sl], y1k)
            ix1 = jnp.maximum(wx1[sl], x1k)
            iy2 = jnp.minimum(wy2[sl], y2k)
            ix2 = jnp.minimum(wx2[sl], x2k)
            inter = jnp.maximum(iy2 - iy1, 0.0) * jnp.maximum(ix2 - ix1, 0.0)
            union = (warea[sl] + area_k) - inter
            w2 = jnp.where(inter + inter > union, NEG, w)
            work[sl] = w2
            idxv = off + lane
            upd = (w2 > mv) | ((w2 == mv) & (idxv < mi))
            return jnp.where(upd, w2, mv), jnp.where(upd, idxv, mi)

        return fuse

    lax.fori_loop(0, MAX_OUT, round_body, (maxv, maxi))

    @pl.when(sid == 0)
    def _():
        pltpu.sync_copy(rec, out_hbm)


@jax.jit
def kernel(scores, boxes, classes):
    n = scores.shape[0]
    nvec = -(-n // (NSUB * LANES))
    per = nvec * LANES
    npad = per * NSUB
    pad = npad - n

    f32 = jnp.float32
    s_p = jnp.concatenate([scores.astype(f32), jnp.full((pad,), -1.0, f32)])
    y1_p = jnp.concatenate([boxes[:, 0].astype(f32), jnp.zeros((pad,), f32)])
    x1_p = jnp.concatenate([boxes[:, 1].astype(f32), jnp.zeros((pad,), f32)])
    y2_p = jnp.concatenate([boxes[:, 2].astype(f32), jnp.zeros((pad,), f32)])
    x2_p = jnp.concatenate([boxes[:, 3].astype(f32), jnp.zeros((pad,), f32)])
    c_p = jnp.concatenate([classes.astype(f32), jnp.zeros((pad,), f32)])

    mesh = plsc.VectorSubcoreMesh(
        core_axis_name="c", subcore_axis_name="s", num_cores=1,
        num_subcores=NSUB)

    run = pl.kernel(
        functools.partial(_nms_body, per, nvec),
        out_type=jax.ShapeDtypeStruct((MAX_OUT * LANES,), f32),
        mesh=mesh,
        compiler_params=pltpu.CompilerParams(needs_layout_passes=False),
        scratch_types=[
            pltpu.VMEM((per,), f32),
            pltpu.VMEM((per,), f32),
            pltpu.VMEM((per,), f32),
            pltpu.VMEM((per,), f32),
            pltpu.VMEM((per,), f32),
            pltpu.VMEM((per,), f32),
            pltpu.VMEM((per,), f32),
            pltpu.VMEM((LANES,), f32),
            pltpu.VMEM((NSUB * ROW,), f32),
            pltpu.VMEM((MAX_OUT * LANES,), f32),
            pltpu.VMEM_SHARED((2 * NSUB * ROW,), f32),
        ],
    )
    rec = run(s_p, y1_p, x1_p, y2_p, x2_p, c_p).reshape(MAX_OUT, LANES)
    return rec[:, 0], rec[:, 1:5], rec[:, 5]


# speedup vs baseline: 13.6369x; 1.0993x over previous
import functools

import jax
import jax.numpy as jnp
from jax import lax
from jax.experimental import pallas as pl
from jax.experimental.pallas import tpu as pltpu
from jax.experimental.pallas import tpu_sc as plsc

LANES = 16
NSUB = 16
MAX_OUT = 100
IOU_THR = 0.5
SCORE_THR = 0.1
NEG = -1.0e30
BIGI = 2**30
BIGF = 1.0e30
ROW = 16


def _nms_body(per, nvec, s_hbm, y1_hbm, x1_hbm, y2_hbm, x2_hbm, c_hbm,
              out_hbm, work, wy1, wx1, wy2, wx2, warea, wcls, stage, buf,
              rec, shared):
    sid = lax.axis_index("s")
    base = sid * per
    lane = lax.iota(jnp.int32, LANES)

    pltpu.sync_copy(s_hbm.at[pl.ds(base, per)], work)
    pltpu.sync_copy(y1_hbm.at[pl.ds(base, per)], wy1)
    pltpu.sync_copy(x1_hbm.at[pl.ds(base, per)], wx1)
    pltpu.sync_copy(y2_hbm.at[pl.ds(base, per)], wy2)
    pltpu.sync_copy(x2_hbm.at[pl.ds(base, per)], wx2)
    pltpu.sync_copy(c_hbm.at[pl.ds(base, per)], wcls)

    def init_body(j, carry):
        maxv, maxi = carry
        off = j * LANES
        v = work[pl.ds(off, LANES)]
        w = jnp.where(v >= SCORE_THR, v, NEG)
        work[pl.ds(off, LANES)] = w
        a = (wy2[pl.ds(off, LANES)] - wy1[pl.ds(off, LANES)]) * (
            wx2[pl.ds(off, LANES)] - wx1[pl.ds(off, LANES)])
        warea[pl.ds(off, LANES)] = a
        idxv = off + lane
        upd = (w > maxv) | ((w == maxv) & (idxv < maxi))
        return jnp.where(upd, w, maxv), jnp.where(upd, idxv, maxi)

    maxv0 = jnp.full((LANES,), NEG, jnp.float32)
    maxi0 = jnp.zeros((LANES,), jnp.int32)
    maxv, maxi = lax.fori_loop(0, nvec, init_body, (maxv0, maxi0))

    def round_body(i, carry):
        maxv, maxi = carry
        par = (i % 2) * (NSUB * ROW)
        m = jnp.max(maxv)
        jloc = jnp.min(jnp.where(maxv == m, maxi, BIGI))
        jidx = jnp.full((LANES,), jloc, jnp.int32)
        y1g = plsc.load_gather(wy1, [jidx])
        x1g = plsc.load_gather(wx1, [jidx])
        y2g = plsc.load_gather(wy2, [jidx])
        x2g = plsc.load_gather(wx2, [jidx])
        cg = plsc.load_gather(wcls, [jidx])
        gidxf = (jloc + base).astype(jnp.float32)
        r = jnp.where(lane == 0, m, 0.0)
        r = jnp.where(lane == 1, gidxf, r)
        r = jnp.where(lane == 2, y1g, r)
        r = jnp.where(lane == 3, x1g, r)
        r = jnp.where(lane == 4, y2g, r)
        r = jnp.where(lane == 5, x2g, r)
        r = jnp.where(lane == 6, cg, r)
        stage[...] = r
        pltpu.sync_copy(stage, shared.at[pl.ds(par + sid * ROW, LANES)])
        plsc.subcore_barrier()
        pltpu.sync_copy(shared.at[pl.ds(par, NSUB * ROW)], buf)

        row = lane * ROW
        maxes = plsc.load_gather(buf, [row])
        idxs = plsc.load_gather(buf, [row + 1])
        m2 = jnp.max(maxes)
        ok = m2 >= SCORE_THR
        cand = jnp.where(maxes == m2, idxs, BIGF)
        widxf = jnp.min(cand)
        widx = widxf.astype(jnp.int32)
        t = widx // per
        trow = jnp.full((LANES,), t * ROW, jnp.int32)
        y1s = plsc.load_gather(buf, [trow + 2])
        x1s = plsc.load_gather(buf, [trow + 3])
        y2s = plsc.load_gather(buf, [trow + 4])
        x2s = plsc.load_gather(buf, [trow + 5])
        cs = plsc.load_gather(buf, [trow + 6])

        rv = jnp.where(lane == 0, m2, 0.0)
        rv = jnp.where(lane == 1, y1s, rv)
        rv = jnp.where(lane == 2, x1s, rv)
        rv = jnp.where(lane == 3, y2s, rv)
        rv = jnp.where(lane == 4, x2s, rv)
        rv = jnp.where(lane == 5, cs, rv)
        rec[pl.ds(i * LANES, LANES)] = jnp.where(ok, rv, 0.0)

        widxi = widx - base
        kmask = ok & (t == sid) & (lane == 0)
        kaddr = jnp.clip(jnp.full((LANES,), widxi, jnp.int32), 0, per - 1)
        plsc.store_scatter(work, [kaddr], jnp.full((LANES,), NEG, jnp.float32),
                           mask=kmask)
        y1k = jnp.where(ok, y1s, BIGF)
        x1k = jnp.where(ok, x1s, BIGF)
        y2k = jnp.where(ok, y2s, BIGF)
        x2k = jnp.where(ok, x2s, BIGF)
        area_k = (y2k - y1k) * (x2k - x1k)

        @plsc.parallel_loop(0, nvec, unroll=8, carry=(maxv0, maxi0))
        def fuse(j, c2):
            mv, mi = c2
            off = (nvec - 1 - j) * LANES
            sl = pl.ds(off, LANES)
            w = work[sl]
            iy1 = jnp.maximum(wy1[sl], y1k)
            ix1 = jnp.maximum(wx1[sl], x1k)
            iy2 = jnp.minimum(wy2[sl], y2k)
            ix2 = jnp.minimum(wx2[sl], x2k)
            inter = jnp.maximum(iy2 - iy1, 0.0) * jnp.maximum(ix2 - ix1, 0.0)
            w2 = jnp.where(3.0 * inter > warea[sl] + area_k, NEG, w)
            work[sl] = w2
            upd = w2 >= mv
            return jnp.where(upd, w2, mv), jnp.where(upd, off + lane, mi)

        return fuse

    lax.fori_loop(0, MAX_OUT, round_body, (maxv, maxi))

    @pl.when(sid == 0)
    def _():
        pltpu.sync_copy(rec, out_hbm)


@jax.jit
def kernel(scores, boxes, classes):
    n = scores.shape[0]
    nvec = -(-n // (NSUB * LANES))
    per = nvec * LANES
    npad = per * NSUB
    pad = npad - n

    f32 = jnp.float32
    s_p = jnp.concatenate([scores.astype(f32), jnp.full((pad,), -1.0, f32)])
    y1_p = jnp.concatenate([boxes[:, 0].astype(f32), jnp.zeros((pad,), f32)])
    x1_p = jnp.concatenate([boxes[:, 1].astype(f32), jnp.zeros((pad,), f32)])
    y2_p = jnp.concatenate([boxes[:, 2].astype(f32), jnp.zeros((pad,), f32)])
    x2_p = jnp.concatenate([boxes[:, 3].astype(f32), jnp.zeros((pad,), f32)])
    c_p = jnp.concatenate([classes.astype(f32), jnp.zeros((pad,), f32)])

    mesh = plsc.VectorSubcoreMesh(
        core_axis_name="c", subcore_axis_name="s", num_cores=1,
        num_subcores=NSUB)

    run = pl.kernel(
        functools.partial(_nms_body, per, nvec),
        out_type=jax.ShapeDtypeStruct((MAX_OUT * LANES,), f32),
        mesh=mesh,
        compiler_params=pltpu.CompilerParams(needs_layout_passes=False),
        scratch_types=[
            pltpu.VMEM((per,), f32),
            pltpu.VMEM((per,), f32),
            pltpu.VMEM((per,), f32),
            pltpu.VMEM((per,), f32),
            pltpu.VMEM((per,), f32),
            pltpu.VMEM((per,), f32),
            pltpu.VMEM((per,), f32),
            pltpu.VMEM((LANES,), f32),
            pltpu.VMEM((NSUB * ROW,), f32),
            pltpu.VMEM((MAX_OUT * LANES,), f32),
            pltpu.VMEM_SHARED((2 * NSUB * ROW,), f32),
        ],
    )
    rec = run(s_p, y1_p, x1_p, y2_p, x2_p, c_p).reshape(MAX_OUT, LANES)
    return rec[:, 0], rec[:, 1:5], rec[:, 5]


# speedup vs baseline: 13.7662x; 1.0095x over previous
import functools

import jax
import jax.numpy as jnp
from jax import lax
from jax.experimental import pallas as pl
from jax.experimental.pallas import tpu as pltpu
from jax.experimental.pallas import tpu_sc as plsc

LANES = 16
NSUB = 16
MAX_OUT = 100
IOU_THR = 0.5
SCORE_THR = 0.1
NEG = -1.0e30
BIGI = 2**30
BIGF = 1.0e30
ROW = 8


def _nms_body(per, nvec, s_hbm, y1_hbm, x1_hbm, y2_hbm, x2_hbm, c_hbm,
              out_hbm, work, wy1, wx1, wy2, wx2, warea, wcls, stage, buf,
              rec, shared):
    sid = lax.axis_index("s")
    base = sid * per
    lane = lax.iota(jnp.int32, LANES)

    pltpu.sync_copy(s_hbm.at[pl.ds(base, per)], work)
    pltpu.sync_copy(y1_hbm.at[pl.ds(base, per)], wy1)
    pltpu.sync_copy(x1_hbm.at[pl.ds(base, per)], wx1)
    pltpu.sync_copy(y2_hbm.at[pl.ds(base, per)], wy2)
    pltpu.sync_copy(x2_hbm.at[pl.ds(base, per)], wx2)
    pltpu.sync_copy(c_hbm.at[pl.ds(base, per)], wcls)

    def init_body(j, carry):
        maxv, maxi = carry
        off = j * LANES
        v = work[pl.ds(off, LANES)]
        w = jnp.where(v >= SCORE_THR, v, NEG)
        work[pl.ds(off, LANES)] = w
        a = (wy2[pl.ds(off, LANES)] - wy1[pl.ds(off, LANES)]) * (
            wx2[pl.ds(off, LANES)] - wx1[pl.ds(off, LANES)])
        warea[pl.ds(off, LANES)] = a
        idxv = off + lane
        upd = (w > maxv) | ((w == maxv) & (idxv < maxi))
        return jnp.where(upd, w, maxv), jnp.where(upd, idxv, maxi)

    maxv0 = jnp.full((LANES,), NEG, jnp.float32)
    maxi0 = jnp.zeros((LANES,), jnp.int32)
    maxv, maxi = lax.fori_loop(0, nvec, init_body, (maxv0, maxi0))

    def round_body(i, carry):
        maxv, maxi = carry
        par = (i % 2) * (NSUB * ROW)
        m = jnp.max(maxv)
        jloc = jnp.min(jnp.where(maxv == m, maxi, BIGI))
        jidx = jnp.full((LANES,), jloc, jnp.int32)
        y1g = plsc.load_gather(wy1, [jidx])
        x1g = plsc.load_gather(wx1, [jidx])
        y2g = plsc.load_gather(wy2, [jidx])
        x2g = plsc.load_gather(wx2, [jidx])
        cg = plsc.load_gather(wcls, [jidx])
        gidxf = (jloc + base).astype(jnp.float32)
        r = jnp.where(lane == 0, m, 0.0)
        r = jnp.where(lane == 1, gidxf, r)
        r = jnp.where(lane == 2, y1g, r)
        r = jnp.where(lane == 3, x1g, r)
        r = jnp.where(lane == 4, y2g, r)
        r = jnp.where(lane == 5, x2g, r)
        r = jnp.where(lane == 6, cg, r)
        stage[...] = r
        pltpu.sync_copy(stage.at[pl.ds(0, ROW)],
                        shared.at[pl.ds(par + sid * ROW, ROW)])
        plsc.subcore_barrier()
        pltpu.sync_copy(shared.at[pl.ds(par, NSUB * ROW)], buf)

        row = lane * ROW
        maxes = plsc.load_gather(buf, [row])
        idxs = plsc.load_gather(buf, [row + 1])
        m2 = jnp.max(maxes)
        ok = m2 >= SCORE_THR
        cand = jnp.where(maxes == m2, idxs, BIGF)
        widxf = jnp.min(cand)
        widx = widxf.astype(jnp.int32)
        t = widx // per
        trow = jnp.full((LANES,), t * ROW, jnp.int32)
        y1s = plsc.load_gather(buf, [trow + 2])
        x1s = plsc.load_gather(buf, [trow + 3])
        y2s = plsc.load_gather(buf, [trow + 4])
        x2s = plsc.load_gather(buf, [trow + 5])
        cs = plsc.load_gather(buf, [trow + 6])

        rv = jnp.where(lane == 0, m2, 0.0)
        rv = jnp.where(lane == 1, y1s, rv)
        rv = jnp.where(lane == 2, x1s, rv)
        rv = jnp.where(lane == 3, y2s, rv)
        rv = jnp.where(lane == 4, x2s, rv)
        rv = jnp.where(lane == 5, cs, rv)
        rec[pl.ds(i * LANES, LANES)] = jnp.where(ok, rv, 0.0)

        widxi = widx - base
        kmask = ok & (t == sid) & (lane == 0)
        kaddr = jnp.clip(jnp.full((LANES,), widxi, jnp.int32), 0, per - 1)
        plsc.store_scatter(work, [kaddr], jnp.full((LANES,), NEG, jnp.float32),
                           mask=kmask)
        y1k = jnp.where(ok, y1s, BIGF)
        x1k = jnp.where(ok, x1s, BIGF)
        y2k = jnp.where(ok, y2s, BIGF)
        x2k = jnp.where(ok, x2s, BIGF)
        area_k = (y2k - y1k) * (x2k - x1k)

        @plsc.parallel_loop(0, nvec, unroll=8, carry=(maxv0, maxi0))
        def fuse(j, c2):
            mv, mi = c2
            off = (nvec - 1 - j) * LANES
            sl = pl.ds(off, LANES)
            w = work[sl]
            iy1 = jnp.maximum(wy1[sl], y1k)
            ix1 = jnp.maximum(wx1[sl], x1k)
            iy2 = jnp.minimum(wy2[sl], y2k)
            ix2 = jnp.minimum(wx2[sl], x2k)
            inter = jnp.maximum(iy2 - iy1, 0.0) * jnp.maximum(ix2 - ix1, 0.0)
            w2 = jnp.where(3.0 * inter > warea[sl] + area_k, NEG, w)
            work[sl] = w2
            upd = w2 >= mv
            return jnp.where(upd, w2, mv), jnp.where(upd, off + lane, mi)

        return fuse

    lax.fori_loop(0, MAX_OUT, round_body, (maxv, maxi))

    @pl.when(sid == 0)
    def _():
        pltpu.sync_copy(rec, out_hbm)


@jax.jit
def kernel(scores, boxes, classes):
    n = scores.shape[0]
    nvec = -(-n // (NSUB * LANES))
    per = nvec * LANES
    npad = per * NSUB
    pad = npad - n

    f32 = jnp.float32
    s_p = jnp.concatenate([scores.astype(f32), jnp.full((pad,), -1.0, f32)])
    y1_p = jnp.concatenate([boxes[:, 0].astype(f32), jnp.zeros((pad,), f32)])
    x1_p = jnp.concatenate([boxes[:, 1].astype(f32), jnp.zeros((pad,), f32)])
    y2_p = jnp.concatenate([boxes[:, 2].astype(f32), jnp.zeros((pad,), f32)])
    x2_p = jnp.concatenate([boxes[:, 3].astype(f32), jnp.zeros((pad,), f32)])
    c_p = jnp.concatenate([classes.astype(f32), jnp.zeros((pad,), f32)])

    mesh = plsc.VectorSubcoreMesh(
        core_axis_name="c", subcore_axis_name="s", num_cores=1,
        num_subcores=NSUB)

    run = pl.kernel(
        functools.partial(_nms_body, per, nvec),
        out_type=jax.ShapeDtypeStruct((MAX_OUT * LANES,), f32),
        mesh=mesh,
        compiler_params=pltpu.CompilerParams(needs_layout_passes=False),
        scratch_types=[
            pltpu.VMEM((per,), f32),
            pltpu.VMEM((per,), f32),
            pltpu.VMEM((per,), f32),
            pltpu.VMEM((per,), f32),
            pltpu.VMEM((per,), f32),
            pltpu.VMEM((per,), f32),
            pltpu.VMEM((per,), f32),
            pltpu.VMEM((LANES,), f32),
            pltpu.VMEM((NSUB * ROW,), f32),
            pltpu.VMEM((MAX_OUT * LANES,), f32),
            pltpu.VMEM_SHARED((2 * NSUB * ROW,), f32),
        ],
    )
    rec = run(s_p, y1_p, x1_p, y2_p, x2_p, c_p).reshape(MAX_OUT, LANES)
    return rec[:, 0], rec[:, 1:5], rec[:, 5]
